# Initial kernel scaffold; baseline (speedup 1.0000x reference)
import jax
import jax.numpy as jnp
from jax.experimental import pallas as pl


def kernel(features, initial_features, edge_index, a_weight):
    raise NotImplementedError("write your pallas kernel here")


# speedup vs baseline: 3.1395x; 3.1395x over previous
import functools

import jax
import jax.numpy as jnp
from jax import lax
from jax.experimental import pallas as pl
from jax.experimental.pallas import tpu as pltpu
from jax.experimental.pallas import tpu_sc as plsc

N = 10000
E = 320000
D = 128

NC = 2
NS = 16
NW = NC * NS

NPAD = 10240
DUMMY = NPAD - 1

E1_PER_TILE = 20480
E1_CHUNKS = E1_PER_TILE // 16

K3_CHUNK = 128
K3_NCHUNK = 80
E3_PER_TILE = K3_CHUNK * K3_NCHUNK

ROWS_PER_TILE = NPAD // NS


def _zero_1d(ref, nwords):
    zeros = jnp.zeros((16,), ref.dtype)

    def body(i, _):
        ref[pl.ds(i * 16, 16)] = zeros
        return _

    lax.fori_loop(0, nwords // 16, body, None)


def _rsqrt16(x):
    i = plsc.bitcast(x, jnp.int32)
    i = 0x5F3759DF - (i >> 1)
    y = plsc.bitcast(i, jnp.float32)
    for _ in range(3):
        y = y * (1.5 - 0.5 * x * y * y)
    return y


def _k1_body(dst_hbm, feat_hbm, norm_hbm, fpre_hbm,
             dstv, histv, scrv, tmpv, accv, normv, featv, deg_sh):
    c = lax.axis_index("c")
    s = lax.axis_index("s")
    w = s * NC + c

    pltpu.sync_copy(dst_hbm.at[s], dstv)

    _zero_1d(histv, NPAD)

    iota = lax.iota(jnp.int32, 16)
    prev_idx = jnp.maximum(iota - 1, 0)
    next_idx = jnp.minimum(iota + 1, 15)

    def chunk_body(j, _):
        d = dstv[pl.ds(j * 16, 16)]
        s_sorted, _v = plsc.sort_key_val(d, d)
        scrv[pl.ds(0, 16)] = s_sorted
        prev = plsc.load_gather(scrv, [prev_idx])
        nxt = plsc.load_gather(scrv, [next_idx])
        first = (s_sorted != prev) | (iota == 0)
        last = (s_sorted != nxt) | (iota == 15)
        fs = plsc.cummax(jnp.where(first, iota, jnp.zeros_like(iota)))
        cnt = (iota - fs + 1).astype(jnp.float32)
        plsc.addupdate_scatter(histv, [s_sorted], cnt, mask=last)
        return _

    lax.fori_loop(0, E1_CHUNKS, chunk_body, None)

    pltpu.sync_copy(histv, deg_sh.at[s])
    plsc.subcore_barrier()

    _zero_1d(accv, ROWS_PER_TILE)
    for t in range(NS):
        pltpu.sync_copy(deg_sh.at[t, pl.ds(s * ROWS_PER_TILE, ROWS_PER_TILE)],
                        tmpv)

        def add_body(u, _):
            accv[pl.ds(u * 16, 16)] = (accv[pl.ds(u * 16, 16)]
                                       + tmpv[pl.ds(u * 16, 16)])
            return _

        lax.fori_loop(0, ROWS_PER_TILE // 16, add_body, None)

    def norm_body(u, _):
        x = jnp.maximum(accv[pl.ds(u * 16, 16)], 1.0)
        normv[pl.ds(u * 16, 16)] = _rsqrt16(x)
        return _

    lax.fori_loop(0, ROWS_PER_TILE // 16, norm_body, None)

    @pl.when(c == 0)
    def _():
        pltpu.sync_copy(normv, norm_hbm.at[pl.ds(s * ROWS_PER_TILE,
                                                 ROWS_PER_TILE)])

    rows = NPAD // NW
    base = w * rows
    pltpu.sync_copy(feat_hbm.at[pl.ds(base, rows)], featv)

    def scale_body(g, _):
        nvv = normv[pl.ds(c * rows + g * 16, 16)]
        for u in range(16):
            nv = nvv[u]
            r = g * 16 + u
            for k in range(D // 16):
                featv[r, pl.ds(k * 16, 16)] = featv[r, pl.ds(k * 16, 16)] * nv
        return _

    lax.fori_loop(0, rows // 16, scale_body, None)
    pltpu.sync_copy(featv, fpre_hbm.at[pl.ds(base, rows)])


def _k3_body(src_hbm, dst_hbm, fpre_hbm, norm_hbm, h0_hbm, h1_hbm,
             srcv, dstv, rowsv, zrow, normv1, acc_sh, gsem):
    c = lax.axis_index("c")
    s = lax.axis_index("s")
    w = s * NC + c

    pltpu.sync_copy(src_hbm.at[w], srcv)
    pltpu.sync_copy(dst_hbm.at[w], dstv)
    pltpu.sync_copy(norm_hbm.at[pl.ds(s * ROWS_PER_TILE, ROWS_PER_TILE)],
                    normv1)

    zeros = jnp.zeros((16,), jnp.float32)
    for i in range(16):
        for k in range(D // 16):
            zrow[i, pl.ds(k * 16, 16)] = zeros

    def zb(b, _):
        pltpu.sync_copy(zrow, acc_sh.at[pl.ds(s * ROWS_PER_TILE + b * 16, 16)])
        return _

    lax.fori_loop(0, ROWS_PER_TILE // 16, zb, None)
    plsc.subcore_barrier()

    def mbody(j, _):
        pltpu.async_copy(fpre_hbm.at[srcv.at[j]], rowsv, gsem).wait()
        pltpu.sync_copy(rowsv, acc_sh.at[dstv.at[j]], add=True)
        return _

    lax.fori_loop(0, K3_NCHUNK, mbody, None)
    plsc.subcore_barrier()

    for cc in range(ROWS_PER_TILE // K3_CHUNK):
        roff = s * ROWS_PER_TILE + cc * K3_CHUNK
        pltpu.sync_copy(acc_sh.at[pl.ds(roff, K3_CHUNK)], rowsv)

        def sc_body(g, _):
            nvv = normv1[pl.ds(cc * K3_CHUNK + g * 16, 16)]
            for u in range(16):
                nv = nvv[u]
                r = g * 16 + u
                for k in range(D // 16):
                    rowsv[r, pl.ds(k * 16, 16)] = (
                        rowsv[r, pl.ds(k * 16, 16)] * nv)
            return _

        lax.fori_loop(0, K3_CHUNK // 16, sc_body, None)

        @pl.when(c == 0)
        def _():
            pltpu.sync_copy(rowsv, h0_hbm.at[pl.ds(roff, K3_CHUNK)])

        @pl.when(c == 1)
        def _():
            pltpu.sync_copy(rowsv, h1_hbm.at[pl.ds(roff, K3_CHUNK)])


def _k4_body(h0_ref, h1_ref, feat_ref, init_ref, aw_ref, out_ref):
    h = h0_ref[...] + h1_ref[...]
    f = feat_ref[...]
    a1 = aw_ref[0:1, 0:D]
    a2 = aw_ref[0:1, D:2 * D]
    logit = (jnp.sum(f * a1, axis=1, keepdims=True)
             + jnp.sum(h * a2, axis=1, keepdims=True))
    alpha = jax.nn.sigmoid(logit)
    out_ref[...] = alpha * h + init_ref[...]


_sc_mesh = plsc.VectorSubcoreMesh(core_axis_name="c", subcore_axis_name="s")

_k1 = functools.partial(
    pl.kernel,
    out_type=(
        jax.ShapeDtypeStruct((NPAD,), jnp.float32),
        jax.ShapeDtypeStruct((NPAD, D), jnp.float32),
    ),
    mesh=_sc_mesh,
    compiler_params=pltpu.CompilerParams(needs_layout_passes=False),
    scratch_types=[
        pltpu.VMEM((E1_PER_TILE,), jnp.int32),
        pltpu.VMEM((NPAD,), jnp.float32),
        pltpu.VMEM((NPAD,), jnp.int32),
        pltpu.VMEM((ROWS_PER_TILE,), jnp.float32),
        pltpu.VMEM((ROWS_PER_TILE,), jnp.float32),
        pltpu.VMEM((ROWS_PER_TILE,), jnp.float32),
        pltpu.VMEM((NPAD // NW, D), jnp.float32),
        pltpu.VMEM_SHARED((NS, NPAD), jnp.float32),
    ],
)(_k1_body)

_k3 = functools.partial(
    pl.kernel,
    out_type=(
        jax.ShapeDtypeStruct((NPAD, D), jnp.float32),
        jax.ShapeDtypeStruct((NPAD, D), jnp.float32),
    ),
    mesh=_sc_mesh,
    scratch_types=[
        pltpu.VMEM((K3_NCHUNK, K3_CHUNK), jnp.int32),
        pltpu.VMEM((K3_NCHUNK, K3_CHUNK), jnp.int32),
        pltpu.VMEM((K3_CHUNK, D), jnp.float32),
        pltpu.VMEM((16, D), jnp.float32),
        pltpu.VMEM((ROWS_PER_TILE,), jnp.float32),
        pltpu.VMEM_SHARED((NPAD, D), jnp.float32),
        pltpu.SemaphoreType.DMA,
    ],
)(_k3_body)

_ROWBLK = 2048
_GRID = 5


def _k4(h0, h1, features, initial_features, a_weight):
    blk = lambda i: (i, 0)
    return pl.pallas_call(
        _k4_body,
        grid=(_GRID,),
        in_specs=[
            pl.BlockSpec((_ROWBLK, D), blk),
            pl.BlockSpec((_ROWBLK, D), blk),
            pl.BlockSpec((_ROWBLK, D), blk),
            pl.BlockSpec((_ROWBLK, D), blk),
            pl.BlockSpec((1, 2 * D), lambda i: (0, 0)),
        ],
        out_specs=pl.BlockSpec((_ROWBLK, D), blk),
        out_shape=jax.ShapeDtypeStruct((N, D), jnp.float32),
    )(h0, h1, features, initial_features, a_weight)


def kernel(features, initial_features, edge_index, a_weight):
    src = edge_index[0]
    dst = edge_index[1]

    pad1 = NS * E1_PER_TILE - E
    dst16 = jnp.concatenate(
        [dst, jnp.full((pad1,), DUMMY, jnp.int32)]).reshape(NS, E1_PER_TILE)

    pad3 = NW * E3_PER_TILE - E
    src3 = jnp.concatenate(
        [src, jnp.zeros((pad3,), jnp.int32)]).reshape(NW, K3_NCHUNK, K3_CHUNK)
    dst3 = jnp.concatenate(
        [dst, jnp.full((pad3,), DUMMY, jnp.int32)]).reshape(
            NW, K3_NCHUNK, K3_CHUNK)

    featp = jnp.pad(features, ((0, NPAD - N), (0, 0)))

    norm, fpre = _k1(dst16, featp)
    h0, h1 = _k3(src3, dst3, fpre, norm)
    out = _k4(h0, h1, features, initial_features, a_weight)
    return out


# speedup vs baseline: 4.2651x; 1.3585x over previous
import functools

import jax
import jax.numpy as jnp
from jax import lax
from jax.experimental import pallas as pl
from jax.experimental.pallas import tpu as pltpu
from jax.experimental.pallas import tpu_sc as plsc

N = 10000
E = 320000
D = 128

NC = 2
NS = 16
NW = NC * NS

NPAD = 10240
DUMMY = NPAD - 1

E1_PER_TILE = 20480
E1_CHUNKS = E1_PER_TILE // 16

K3_CHUNK = 128
K3_NCHUNK = 80
E3_PER_TILE = K3_CHUNK * K3_NCHUNK

ROWS_PER_TILE = NPAD // NS


def _zero_1d(ref, nwords):
    zeros = jnp.zeros((16,), ref.dtype)

    def body(i, _):
        ref[pl.ds(i * 16, 16)] = zeros
        return _

    lax.fori_loop(0, nwords // 16, body, None)


def _rsqrt16(x):
    i = plsc.bitcast(x, jnp.int32)
    i = 0x5F3759DF - (i >> 1)
    y = plsc.bitcast(i, jnp.float32)
    for _ in range(3):
        y = y * (1.5 - 0.5 * x * y * y)
    return y


def _k1_body(dst_hbm, feat_hbm, norm_hbm, fpre_hbm,
             dstv, histv, scrv, tmpv, accv, normv, featv, deg_sh):
    c = lax.axis_index("c")
    s = lax.axis_index("s")
    w = s * NC + c

    pltpu.sync_copy(dst_hbm.at[s], dstv)

    _zero_1d(histv, NPAD)

    iota = lax.iota(jnp.int32, 16)
    prev_idx = jnp.maximum(iota - 1, 0)
    next_idx = jnp.minimum(iota + 1, 15)

    def chunk_body(j, _):
        d = dstv[pl.ds(j * 16, 16)]
        s_sorted, _v = plsc.sort_key_val(d, d)
        scrv[pl.ds(0, 16)] = s_sorted
        prev = plsc.load_gather(scrv, [prev_idx])
        nxt = plsc.load_gather(scrv, [next_idx])
        first = (s_sorted != prev) | (iota == 0)
        last = (s_sorted != nxt) | (iota == 15)
        fs = plsc.cummax(jnp.where(first, iota, jnp.zeros_like(iota)))
        cnt = (iota - fs + 1).astype(jnp.float32)
        plsc.addupdate_scatter(histv, [s_sorted], cnt, mask=last)
        return _

    lax.fori_loop(0, E1_CHUNKS, chunk_body, None)

    pltpu.sync_copy(histv, deg_sh.at[s])
    plsc.subcore_barrier()

    _zero_1d(accv, ROWS_PER_TILE)
    for t in range(NS):
        pltpu.sync_copy(deg_sh.at[t, pl.ds(s * ROWS_PER_TILE, ROWS_PER_TILE)],
                        tmpv)

        def add_body(u, _):
            accv[pl.ds(u * 16, 16)] = (accv[pl.ds(u * 16, 16)]
                                       + tmpv[pl.ds(u * 16, 16)])
            return _

        lax.fori_loop(0, ROWS_PER_TILE // 16, add_body, None)

    def norm_body(u, _):
        x = jnp.maximum(accv[pl.ds(u * 16, 16)], 1.0)
        normv[pl.ds(u * 16, 16)] = _rsqrt16(x)
        return _

    lax.fori_loop(0, ROWS_PER_TILE // 16, norm_body, None)

    @pl.when(c == 0)
    def _():
        pltpu.sync_copy(normv, norm_hbm.at[pl.ds(s * ROWS_PER_TILE,
                                                 ROWS_PER_TILE)])

    rows = NPAD // NW
    base = w * rows
    pltpu.sync_copy(feat_hbm.at[pl.ds(base, rows)], featv)

    def scale_body(g, _):
        nvv = normv[pl.ds(c * rows + g * 16, 16)]
        for u in range(16):
            nv = nvv[u]
            r = g * 16 + u
            for k in range(D // 16):
                featv[r, pl.ds(k * 16, 16)] = featv[r, pl.ds(k * 16, 16)] * nv
        return _

    lax.fori_loop(0, rows // 16, scale_body, None)
    pltpu.sync_copy(featv, fpre_hbm.at[pl.ds(base, rows)])


def _k3_body(src_hbm, dst_hbm, fpre_hbm, norm_hbm, h0_hbm, h1_hbm,
             sidx, didx, rowsv, rows2v, zrow, normv1, acc_sh, gsem, gsem2):
    c = lax.axis_index("c")
    s = lax.axis_index("s")
    w = s * NC + c

    pltpu.sync_copy(norm_hbm.at[pl.ds(s * ROWS_PER_TILE, ROWS_PER_TILE)],
                    normv1)

    zeros = jnp.zeros((16,), jnp.float32)
    for i in range(16):
        for k in range(D // 16):
            zrow[i, pl.ds(k * 16, 16)] = zeros

    def zb(b, _):
        pltpu.sync_copy(zrow, acc_sh.at[pl.ds(s * ROWS_PER_TILE + b * 16, 16)])
        return _

    lax.fori_loop(0, ROWS_PER_TILE // 16, zb, None)
    plsc.subcore_barrier()

    BLK = 16
    for b in range(K3_NCHUNK // BLK):
        pltpu.sync_copy(src_hbm.at[w, pl.ds(b * BLK, BLK)], sidx)
        pltpu.sync_copy(dst_hbm.at[w, pl.ds(b * BLK, BLK)], didx)
        pltpu.async_copy(fpre_hbm.at[sidx.at[0]], rowsv, gsem)
        pltpu.async_copy(fpre_hbm.at[sidx.at[1]], rows2v, gsem2)

        def mbody(ii, _):
            i0 = 2 * ii
            pltpu.make_async_copy(fpre_hbm.at[sidx.at[i0]], rowsv,
                                  gsem).wait()
            pltpu.sync_copy(rowsv, acc_sh.at[didx.at[i0]], add=True)
            pltpu.async_copy(fpre_hbm.at[sidx.at[i0 + 2]], rowsv, gsem)
            pltpu.make_async_copy(fpre_hbm.at[sidx.at[i0 + 1]], rows2v,
                                  gsem2).wait()
            pltpu.sync_copy(rows2v, acc_sh.at[didx.at[i0 + 1]], add=True)
            pltpu.async_copy(fpre_hbm.at[sidx.at[i0 + 3]], rows2v, gsem2)
            return _

        lax.fori_loop(0, BLK // 2 - 1, mbody, None)
        ilast = BLK - 2
        pltpu.make_async_copy(fpre_hbm.at[sidx.at[ilast]], rowsv,
                              gsem).wait()
        pltpu.sync_copy(rowsv, acc_sh.at[didx.at[ilast]], add=True)
        pltpu.make_async_copy(fpre_hbm.at[sidx.at[ilast + 1]], rows2v,
                              gsem2).wait()
        pltpu.sync_copy(rows2v, acc_sh.at[didx.at[ilast + 1]], add=True)
    plsc.subcore_barrier()

    for cc in range(ROWS_PER_TILE // K3_CHUNK):
        roff = s * ROWS_PER_TILE + cc * K3_CHUNK
        pltpu.sync_copy(acc_sh.at[pl.ds(roff, K3_CHUNK)], rowsv)

        def sc_body(g, _):
            nvv = normv1[pl.ds(cc * K3_CHUNK + g * 16, 16)]
            for u in range(16):
                nv = nvv[u]
                r = g * 16 + u
                for k in range(D // 16):
                    rowsv[r, pl.ds(k * 16, 16)] = (
                        rowsv[r, pl.ds(k * 16, 16)] * nv)
            return _

        lax.fori_loop(0, K3_CHUNK // 16, sc_body, None)

        @pl.when(c == 0)
        def _():
            pltpu.sync_copy(rowsv, h0_hbm.at[pl.ds(roff, K3_CHUNK)])

        @pl.when(c == 1)
        def _():
            pltpu.sync_copy(rowsv, h1_hbm.at[pl.ds(roff, K3_CHUNK)])


def _k4_body(h0_ref, h1_ref, feat_ref, init_ref, aw_ref, out_ref):
    h = h0_ref[...] + h1_ref[...]
    f = feat_ref[...]
    a1 = aw_ref[0:1, 0:D]
    a2 = aw_ref[0:1, D:2 * D]
    logit = (jnp.sum(f * a1, axis=1, keepdims=True)
             + jnp.sum(h * a2, axis=1, keepdims=True))
    alpha = jax.nn.sigmoid(logit)
    out_ref[...] = alpha * h + init_ref[...]


_sc_mesh = plsc.VectorSubcoreMesh(core_axis_name="c", subcore_axis_name="s")

_k1 = functools.partial(
    pl.kernel,
    out_type=(
        jax.ShapeDtypeStruct((NPAD,), jnp.float32),
        jax.ShapeDtypeStruct((NPAD, D), jnp.float32),
    ),
    mesh=_sc_mesh,
    compiler_params=pltpu.CompilerParams(needs_layout_passes=False),
    scratch_types=[
        pltpu.VMEM((E1_PER_TILE,), jnp.int32),
        pltpu.VMEM((NPAD,), jnp.float32),
        pltpu.VMEM((NPAD,), jnp.int32),
        pltpu.VMEM((ROWS_PER_TILE,), jnp.float32),
        pltpu.VMEM((ROWS_PER_TILE,), jnp.float32),
        pltpu.VMEM((ROWS_PER_TILE,), jnp.float32),
        pltpu.VMEM((NPAD // NW, D), jnp.float32),
        pltpu.VMEM_SHARED((NS, NPAD), jnp.float32),
    ],
)(_k1_body)

_k3 = functools.partial(
    pl.kernel,
    out_type=(
        jax.ShapeDtypeStruct((NPAD, D), jnp.float32),
        jax.ShapeDtypeStruct((NPAD, D), jnp.float32),
    ),
    mesh=_sc_mesh,
    scratch_types=[
        pltpu.VMEM((16, K3_CHUNK), jnp.int32),
        pltpu.VMEM((16, K3_CHUNK), jnp.int32),
        pltpu.VMEM((K3_CHUNK, D), jnp.float32),
        pltpu.VMEM((K3_CHUNK, D), jnp.float32),
        pltpu.VMEM((16, D), jnp.float32),
        pltpu.VMEM((ROWS_PER_TILE,), jnp.float32),
        pltpu.VMEM_SHARED((NPAD, D), jnp.float32),
        pltpu.SemaphoreType.DMA,
        pltpu.SemaphoreType.DMA,
    ],
)(_k3_body)

_ROWBLK = 2048
_GRID = 5


def _k4(h0, h1, features, initial_features, a_weight):
    blk = lambda i: (i, 0)
    return pl.pallas_call(
        _k4_body,
        grid=(_GRID,),
        in_specs=[
            pl.BlockSpec((_ROWBLK, D), blk),
            pl.BlockSpec((_ROWBLK, D), blk),
            pl.BlockSpec((_ROWBLK, D), blk),
            pl.BlockSpec((_ROWBLK, D), blk),
            pl.BlockSpec((1, 2 * D), lambda i: (0, 0)),
        ],
        out_specs=pl.BlockSpec((_ROWBLK, D), blk),
        out_shape=jax.ShapeDtypeStruct((N, D), jnp.float32),
    )(h0, h1, features, initial_features, a_weight)


def kernel(features, initial_features, edge_index, a_weight):
    src = edge_index[0]
    dst = edge_index[1]

    pad1 = NS * E1_PER_TILE - E
    dst16 = jnp.concatenate(
        [dst, jnp.full((pad1,), DUMMY, jnp.int32)]).reshape(NS, E1_PER_TILE)

    pad3 = NW * E3_PER_TILE - E
    dummy_dst = N + jnp.arange(pad3, dtype=jnp.int32) % (NPAD - N)
    src3 = jnp.concatenate(
        [src, jnp.zeros((pad3,), jnp.int32)]).reshape(NW, K3_NCHUNK, K3_CHUNK)
    dst3 = jnp.concatenate(
        [dst, dummy_dst]).reshape(NW, K3_NCHUNK, K3_CHUNK)

    featp = jnp.pad(features, ((0, NPAD - N), (0, 0)))

    norm, fpre = _k1(dst16, featp)
    h0, h1 = _k3(src3, dst3, fpre, norm)
    out = _k4(h0, h1, features, initial_features, a_weight)
    return out


# speedup vs baseline: 9.8668x; 2.3134x over previous
import functools

import jax
import jax.numpy as jnp
from jax import lax
from jax.experimental import pallas as pl
from jax.experimental.pallas import tpu as pltpu
from jax.experimental.pallas import tpu_sc as plsc

N = 10000
E = 320000
D = 128

NC = 2
NS = 16
NW = NC * NS

NPAD = 10240
DUMMY = NPAD - 1

E1_PER_TILE = 20480
E1_CHUNKS = E1_PER_TILE // 16

K3_CHUNK = 128
K3_NCHUNK = 80
E3_PER_TILE = K3_CHUNK * K3_NCHUNK

ROWS_PER_TILE = NPAD // NS


def _zero_1d(ref, nwords):
    zeros = jnp.zeros((16,), ref.dtype)

    def body(i, _):
        ref[pl.ds(i * 16, 16)] = zeros
        return _

    lax.fori_loop(0, nwords // 16, body, None)


def _rsqrt16(x):
    i = plsc.bitcast(x, jnp.int32)
    i = 0x5F3759DF - (i >> 1)
    y = plsc.bitcast(i, jnp.float32)
    for _ in range(3):
        y = y * (1.5 - 0.5 * x * y * y)
    return y


def _k1_body(dst_hbm, feat_hbm, norm_hbm, fpre_hbm,
             dstv, histv, scrv, tmpv, accv, normv, featv, deg_sh):
    c = lax.axis_index("c")
    s = lax.axis_index("s")
    w = s * NC + c

    pltpu.sync_copy(dst_hbm.at[s], dstv)

    _zero_1d(histv, NPAD)

    iota = lax.iota(jnp.int32, 16)
    prev_idx = jnp.maximum(iota - 1, 0)
    next_idx = jnp.minimum(iota + 1, 15)

    def chunk_body(j, _):
        d = dstv[pl.ds(j * 16, 16)]
        s_sorted, _v = plsc.sort_key_val(d, d)
        scrv[pl.ds(0, 16)] = s_sorted
        prev = plsc.load_gather(scrv, [prev_idx])
        nxt = plsc.load_gather(scrv, [next_idx])
        first = (s_sorted != prev) | (iota == 0)
        last = (s_sorted != nxt) | (iota == 15)
        fs = plsc.cummax(jnp.where(first, iota, jnp.zeros_like(iota)))
        cnt = (iota - fs + 1).astype(jnp.float32)
        plsc.addupdate_scatter(histv, [s_sorted], cnt, mask=last)
        return _

    lax.fori_loop(0, E1_CHUNKS, chunk_body, None)

    pltpu.sync_copy(histv, deg_sh.at[s])
    plsc.subcore_barrier()

    _zero_1d(accv, ROWS_PER_TILE)
    for t in range(NS):
        pltpu.sync_copy(deg_sh.at[t, pl.ds(s * ROWS_PER_TILE, ROWS_PER_TILE)],
                        tmpv)

        def add_body(u, _):
            accv[pl.ds(u * 16, 16)] = (accv[pl.ds(u * 16, 16)]
                                       + tmpv[pl.ds(u * 16, 16)])
            return _

        lax.fori_loop(0, ROWS_PER_TILE // 16, add_body, None)

    def norm_body(u, _):
        x = jnp.maximum(accv[pl.ds(u * 16, 16)], 1.0)
        normv[pl.ds(u * 16, 16)] = _rsqrt16(x)
        return _

    lax.fori_loop(0, ROWS_PER_TILE // 16, norm_body, None)

    @pl.when(c == 0)
    def _():
        pltpu.sync_copy(normv, norm_hbm.at[pl.ds(s * ROWS_PER_TILE,
                                                 ROWS_PER_TILE)])

    rows = NPAD // NW
    base = w * rows
    pltpu.sync_copy(feat_hbm.at[pl.ds(base, rows)], featv)

    def scale_body(g, _):
        nvv = normv[pl.ds(c * rows + g * 16, 16)]
        for u in range(16):
            nv = nvv[u]
            r = g * 16 + u
            for k in range(D // 16):
                featv[r, pl.ds(k * 16, 16)] = featv[r, pl.ds(k * 16, 16)] * nv
        return _

    lax.fori_loop(0, rows // 16, scale_body, None)
    pltpu.sync_copy(featv, fpre_hbm.at[pl.ds(base, rows)])


def _k3_body(src_hbm, dst_hbm, fpre_hbm, norm_hbm, h0_hbm, h1_hbm,
             sidx, didx, rowsv, rows2v, zrow, normv1, acc_sh, gsem, gsem2):
    c = lax.axis_index("c")
    s = lax.axis_index("s")
    w = s * NC + c

    pltpu.sync_copy(norm_hbm.at[pl.ds(s * ROWS_PER_TILE, ROWS_PER_TILE)],
                    normv1)

    zeros = jnp.zeros((16,), jnp.float32)
    for i in range(16):
        for k in range(D // 16):
            zrow[i, pl.ds(k * 16, 16)] = zeros

    def zb(b, _):
        pltpu.sync_copy(zrow, acc_sh.at[pl.ds(s * ROWS_PER_TILE + b * 16, 16)])
        return _

    lax.fori_loop(0, ROWS_PER_TILE // 16, zb, None)
    plsc.subcore_barrier()

    BLK = 16
    for b in range(K3_NCHUNK // BLK):
        pltpu.sync_copy(src_hbm.at[w, pl.ds(b * BLK, BLK)], sidx)
        pltpu.sync_copy(dst_hbm.at[w, pl.ds(b * BLK, BLK)], didx)
        pltpu.async_copy(fpre_hbm.at[sidx.at[0]], rowsv, gsem)
        pltpu.async_copy(fpre_hbm.at[sidx.at[1]], rows2v, gsem2)

        def mbody(ii, _):
            i0 = 2 * ii
            pltpu.make_async_copy(fpre_hbm.at[sidx.at[i0]], rowsv,
                                  gsem).wait()
            pltpu.sync_copy(rowsv, acc_sh.at[didx.at[i0]], add=True)
            pltpu.async_copy(fpre_hbm.at[sidx.at[i0 + 2]], rowsv, gsem)
            pltpu.make_async_copy(fpre_hbm.at[sidx.at[i0 + 1]], rows2v,
                                  gsem2).wait()
            pltpu.sync_copy(rows2v, acc_sh.at[didx.at[i0 + 1]], add=True)
            pltpu.async_copy(fpre_hbm.at[sidx.at[i0 + 3]], rows2v, gsem2)
            return _

        lax.fori_loop(0, BLK // 2 - 1, mbody, None)
        ilast = BLK - 2
        pltpu.make_async_copy(fpre_hbm.at[sidx.at[ilast]], rowsv,
                              gsem).wait()
        pltpu.sync_copy(rowsv, acc_sh.at[didx.at[ilast]], add=True)
        pltpu.make_async_copy(fpre_hbm.at[sidx.at[ilast + 1]], rows2v,
                              gsem2).wait()
        pltpu.sync_copy(rows2v, acc_sh.at[didx.at[ilast + 1]], add=True)
    plsc.subcore_barrier()

    for cc in range(ROWS_PER_TILE // K3_CHUNK):
        roff = s * ROWS_PER_TILE + cc * K3_CHUNK
        pltpu.sync_copy(acc_sh.at[pl.ds(roff, K3_CHUNK)], rowsv)

        def sc_body(g, _):
            nvv = normv1[pl.ds(cc * K3_CHUNK + g * 16, 16)]
            for u in range(16):
                nv = nvv[u]
                r = g * 16 + u
                for k in range(D // 16):
                    rowsv[r, pl.ds(k * 16, 16)] = (
                        rowsv[r, pl.ds(k * 16, 16)] * nv)
            return _

        lax.fori_loop(0, K3_CHUNK // 16, sc_body, None)

        @pl.when(c == 0)
        def _():
            pltpu.sync_copy(rowsv, h0_hbm.at[pl.ds(roff, K3_CHUNK)])

        @pl.when(c == 1)
        def _():
            pltpu.sync_copy(rowsv, h1_hbm.at[pl.ds(roff, K3_CHUNK)])


def _k4_body(h0_ref, h1_ref, feat_ref, init_ref, aw_ref, out_ref):
    h = h0_ref[...] + h1_ref[...]
    f = feat_ref[...]
    a1 = aw_ref[0:1, 0:D]
    a2 = aw_ref[0:1, D:2 * D]
    logit = (jnp.sum(f * a1, axis=1, keepdims=True)
             + jnp.sum(h * a2, axis=1, keepdims=True))
    alpha = jax.nn.sigmoid(logit)
    out_ref[...] = alpha * h + init_ref[...]


_sc_mesh = plsc.VectorSubcoreMesh(core_axis_name="c", subcore_axis_name="s")

_k1 = functools.partial(
    pl.kernel,
    out_type=(
        jax.ShapeDtypeStruct((NPAD,), jnp.float32),
        jax.ShapeDtypeStruct((NPAD, D), jnp.float32),
    ),
    mesh=_sc_mesh,
    compiler_params=pltpu.CompilerParams(needs_layout_passes=False),
    scratch_types=[
        pltpu.VMEM((E1_PER_TILE,), jnp.int32),
        pltpu.VMEM((NPAD,), jnp.float32),
        pltpu.VMEM((NPAD,), jnp.int32),
        pltpu.VMEM((ROWS_PER_TILE,), jnp.float32),
        pltpu.VMEM((ROWS_PER_TILE,), jnp.float32),
        pltpu.VMEM((ROWS_PER_TILE,), jnp.float32),
        pltpu.VMEM((NPAD // NW, D), jnp.float32),
        pltpu.VMEM_SHARED((NS, NPAD), jnp.float32),
    ],
)(_k1_body)

_k3 = functools.partial(
    pl.kernel,
    out_type=(
        jax.ShapeDtypeStruct((NPAD, D), jnp.float32),
        jax.ShapeDtypeStruct((NPAD, D), jnp.float32),
    ),
    mesh=_sc_mesh,
    scratch_types=[
        pltpu.VMEM((16, K3_CHUNK), jnp.int32),
        pltpu.VMEM((16, K3_CHUNK), jnp.int32),
        pltpu.VMEM((K3_CHUNK, D), jnp.float32),
        pltpu.VMEM((K3_CHUNK, D), jnp.float32),
        pltpu.VMEM((16, D), jnp.float32),
        pltpu.VMEM((ROWS_PER_TILE,), jnp.float32),
        pltpu.VMEM_SHARED((NPAD, D), jnp.float32),
        pltpu.SemaphoreType.DMA,
        pltpu.SemaphoreType.DMA,
    ],
)(_k3_body)

_ROWBLK = 2048
_GRID = 5


def _k4(h0, h1, features, initial_features, a_weight):
    blk = lambda i: (i, 0)
    return pl.pallas_call(
        _k4_body,
        grid=(_GRID,),
        in_specs=[
            pl.BlockSpec((_ROWBLK, D), blk),
            pl.BlockSpec((_ROWBLK, D), blk),
            pl.BlockSpec((_ROWBLK, D), blk),
            pl.BlockSpec((_ROWBLK, D), blk),
            pl.BlockSpec((1, 2 * D), lambda i: (0, 0)),
        ],
        out_specs=pl.BlockSpec((_ROWBLK, D), blk),
        out_shape=jax.ShapeDtypeStruct((N, D), jnp.float32),
    )(h0, h1, features, initial_features, a_weight)


def kernel(features, initial_features, edge_index, a_weight):
    src = edge_index[0]
    dst = edge_index[1]

    pad1 = NS * E1_PER_TILE - E
    dst16 = jnp.concatenate(
        [dst, jnp.full((pad1,), DUMMY, jnp.int32)]).reshape(NS, E1_PER_TILE)

    pad3 = NW * E3_PER_TILE - E
    dummy_dst = N + jnp.arange(pad3, dtype=jnp.int32) % (NPAD - N)
    dummy_src = jnp.arange(pad3, dtype=jnp.int32) % N
    src3 = jnp.concatenate(
        [src, dummy_src]).reshape(NW, K3_NCHUNK, K3_CHUNK)
    dst3 = jnp.concatenate(
        [dst, dummy_dst]).reshape(NW, K3_NCHUNK, K3_CHUNK)

    featp = jnp.pad(features, ((0, NPAD - N), (0, 0)))

    norm, fpre = _k1(dst16, featp)
    h0, h1 = _k3(src3, dst3, fpre, norm)
    out = _k4(h0, h1, features, initial_features, a_weight)
    return out


# speedup vs baseline: 11.4807x; 1.1636x over previous
import functools

import jax
import jax.numpy as jnp
from jax import lax
from jax.experimental import pallas as pl
from jax.experimental.pallas import tpu as pltpu
from jax.experimental.pallas import tpu_sc as plsc

N = 10000
E = 320000
D = 128

NC = 2
NS = 16
NW = NC * NS

NPAD = 10240

K3_CHUNK = 128
K3_NCHUNK = 80
E3_PER_TILE = K3_CHUNK * K3_NCHUNK

ROWS_PER_TILE = NPAD // NS


def _zero_1d(ref, nwords):
    zeros = jnp.zeros((16,), ref.dtype)

    def body(i, _):
        ref[pl.ds(i * 16, 16)] = zeros
        return _

    lax.fori_loop(0, nwords // 16, body, None)


def _k1_body(dst_hbm, hist_hbm, dstv, histv, scrv):
    c = lax.axis_index("c")
    s = lax.axis_index("s")
    w = s * NC + c

    pltpu.sync_copy(dst_hbm.at[w], dstv)
    _zero_1d(histv, NPAD)

    iota = lax.iota(jnp.int32, 16)
    prev_idx = jnp.maximum(iota - 1, 0)
    next_idx = jnp.minimum(iota + 1, 15)

    def chunk_body(j, _):
        for k in range(K3_CHUNK // 16):
            d = dstv[j, pl.ds(k * 16, 16)]
            s_sorted, _v = plsc.sort_key_val(d, d)
            scrv[pl.ds(0, 16)] = s_sorted
            prev = plsc.load_gather(scrv, [prev_idx])
            nxt = plsc.load_gather(scrv, [next_idx])
            first = (s_sorted != prev) | (iota == 0)
            last = (s_sorted != nxt) | (iota == 15)
            fs = plsc.cummax(jnp.where(first, iota, jnp.zeros_like(iota)))
            cnt = (iota - fs + 1).astype(jnp.float32)
            plsc.addupdate_scatter(histv, [s_sorted], cnt, mask=last)
        return _

    lax.fori_loop(0, K3_NCHUNK, chunk_body, None)
    pltpu.sync_copy(histv, hist_hbm.at[w])


def _k3_body(src_hbm, dst_hbm, fpre_hbm, h0_hbm, h1_hbm,
             sidx, didx, rowsv, rows2v, zrow, acc_sh, gsem, gsem2):
    c = lax.axis_index("c")
    s = lax.axis_index("s")
    w = s * NC + c

    zeros = jnp.zeros((16,), jnp.float32)
    for i in range(16):
        for k in range(D // 16):
            zrow[i, pl.ds(k * 16, 16)] = zeros

    def zb(b, _):
        pltpu.sync_copy(zrow, acc_sh.at[pl.ds(s * ROWS_PER_TILE + b * 16, 16)])
        return _

    lax.fori_loop(0, ROWS_PER_TILE // 16, zb, None)
    plsc.subcore_barrier()

    BLK = 16
    for b in range(K3_NCHUNK // BLK):
        pltpu.sync_copy(src_hbm.at[w, pl.ds(b * BLK, BLK)], sidx)
        pltpu.sync_copy(dst_hbm.at[w, pl.ds(b * BLK, BLK)], didx)
        pltpu.async_copy(fpre_hbm.at[sidx.at[0]], rowsv, gsem)
        pltpu.async_copy(fpre_hbm.at[sidx.at[1]], rows2v, gsem2)

        def mbody(ii, _):
            i0 = 2 * ii
            pltpu.make_async_copy(fpre_hbm.at[sidx.at[i0]], rowsv,
                                  gsem).wait()
            pltpu.sync_copy(rowsv, acc_sh.at[didx.at[i0]], add=True)
            pltpu.async_copy(fpre_hbm.at[sidx.at[i0 + 2]], rowsv, gsem)
            pltpu.make_async_copy(fpre_hbm.at[sidx.at[i0 + 1]], rows2v,
                                  gsem2).wait()
            pltpu.sync_copy(rows2v, acc_sh.at[didx.at[i0 + 1]], add=True)
            pltpu.async_copy(fpre_hbm.at[sidx.at[i0 + 3]], rows2v, gsem2)
            return _

        lax.fori_loop(0, BLK // 2 - 1, mbody, None)
        ilast = BLK - 2
        pltpu.make_async_copy(fpre_hbm.at[sidx.at[ilast]], rowsv,
                              gsem).wait()
        pltpu.sync_copy(rowsv, acc_sh.at[didx.at[ilast]], add=True)
        pltpu.make_async_copy(fpre_hbm.at[sidx.at[ilast + 1]], rows2v,
                              gsem2).wait()
        pltpu.sync_copy(rows2v, acc_sh.at[didx.at[ilast + 1]], add=True)
    plsc.subcore_barrier()

    for cc in range(ROWS_PER_TILE // K3_CHUNK):
        roff = s * ROWS_PER_TILE + cc * K3_CHUNK
        pltpu.sync_copy(acc_sh.at[pl.ds(roff, K3_CHUNK)], rowsv)

        @pl.when(c == 0)
        def _():
            pltpu.sync_copy(rowsv, h0_hbm.at[pl.ds(roff, K3_CHUNK)])

        @pl.when(c == 1)
        def _():
            pltpu.sync_copy(rowsv, h1_hbm.at[pl.ds(roff, K3_CHUNK)])


_ROWBLK = 2048
_GRID = NPAD // _ROWBLK


def _norm_col(norm2d):
    nq = _ROWBLK // D
    e1 = (lax.broadcasted_iota(jnp.int32, (_ROWBLK, nq), 1)
          == lax.broadcasted_iota(jnp.int32, (_ROWBLK, nq), 0) // D
          ).astype(jnp.float32)
    t = jnp.dot(e1, norm2d, preferred_element_type=jnp.float32)
    sel = (lax.broadcasted_iota(jnp.int32, (_ROWBLK, D), 1)
           == lax.broadcasted_iota(jnp.int32, (_ROWBLK, D), 0) % D
           ).astype(jnp.float32)
    return jnp.sum(t * sel, axis=1, keepdims=True)


def _k2_body(hist_ref, feat_ref, fpre_ref, norm_ref):
    deg = jnp.sum(hist_ref[...], axis=0)
    norm2d = lax.rsqrt(jnp.maximum(deg, 1.0))
    norm_ref[...] = norm2d
    fpre_ref[...] = feat_ref[...] * _norm_col(norm2d)


def _k2(hist3, features):
    return pl.pallas_call(
        _k2_body,
        grid=(_GRID,),
        in_specs=[
            pl.BlockSpec((NW, _ROWBLK // D, D), lambda i: (0, i, 0)),
            pl.BlockSpec((_ROWBLK, D), lambda i: (i, 0)),
        ],
        out_specs=[
            pl.BlockSpec((_ROWBLK, D), lambda i: (i, 0)),
            pl.BlockSpec((_ROWBLK // D, D), lambda i: (i, 0)),
        ],
        out_shape=[
            jax.ShapeDtypeStruct((NPAD, D), jnp.float32),
            jax.ShapeDtypeStruct((NPAD // D, D), jnp.float32),
        ],
    )(hist3, features)


def _k4_body(h0_ref, h1_ref, norm_ref, feat_ref, init_ref, aw_ref, out_ref):
    h = (h0_ref[...] + h1_ref[...]) * _norm_col(norm_ref[...])
    f = feat_ref[...]
    a1 = aw_ref[0:1, 0:D]
    a2 = aw_ref[0:1, D:2 * D]
    logit = (jnp.sum(f * a1, axis=1, keepdims=True)
             + jnp.sum(h * a2, axis=1, keepdims=True))
    alpha = jax.nn.sigmoid(logit)
    out_ref[...] = alpha * h + init_ref[...]


def _k4(h0, h1, norm2d, features, initial_features, a_weight):
    blk = lambda i: (i, 0)
    return pl.pallas_call(
        _k4_body,
        grid=(_GRID,),
        in_specs=[
            pl.BlockSpec((_ROWBLK, D), blk),
            pl.BlockSpec((_ROWBLK, D), blk),
            pl.BlockSpec((_ROWBLK // D, D), blk),
            pl.BlockSpec((_ROWBLK, D), blk),
            pl.BlockSpec((_ROWBLK, D), blk),
            pl.BlockSpec((1, 2 * D), lambda i: (0, 0)),
        ],
        out_specs=pl.BlockSpec((_ROWBLK, D), blk),
        out_shape=jax.ShapeDtypeStruct((N, D), jnp.float32),
    )(h0, h1, norm2d, features, initial_features, a_weight)


_sc_mesh = plsc.VectorSubcoreMesh(core_axis_name="c", subcore_axis_name="s")

_k1 = functools.partial(
    pl.kernel,
    out_type=jax.ShapeDtypeStruct((NW, NPAD), jnp.float32),
    mesh=_sc_mesh,
    compiler_params=pltpu.CompilerParams(needs_layout_passes=False),
    scratch_types=[
        pltpu.VMEM((K3_NCHUNK, K3_CHUNK), jnp.int32),
        pltpu.VMEM((NPAD,), jnp.float32),
        pltpu.VMEM((16,), jnp.int32),
    ],
)(_k1_body)

_k3 = functools.partial(
    pl.kernel,
    out_type=(
        jax.ShapeDtypeStruct((NPAD, D), jnp.float32),
        jax.ShapeDtypeStruct((NPAD, D), jnp.float32),
    ),
    mesh=_sc_mesh,
    scratch_types=[
        pltpu.VMEM((16, K3_CHUNK), jnp.int32),
        pltpu.VMEM((16, K3_CHUNK), jnp.int32),
        pltpu.VMEM((K3_CHUNK, D), jnp.float32),
        pltpu.VMEM((K3_CHUNK, D), jnp.float32),
        pltpu.VMEM((16, D), jnp.float32),
        pltpu.VMEM_SHARED((NPAD, D), jnp.float32),
        pltpu.SemaphoreType.DMA,
        pltpu.SemaphoreType.DMA,
    ],
)(_k3_body)


def kernel(features, initial_features, edge_index, a_weight):
    src = edge_index[0]
    dst = edge_index[1]

    pad3 = NW * E3_PER_TILE - E
    dummy_dst = N + jnp.arange(pad3, dtype=jnp.int32) % (NPAD - N)
    dummy_src = jnp.arange(pad3, dtype=jnp.int32) % N
    src3 = jnp.concatenate(
        [src, dummy_src]).reshape(NW, K3_NCHUNK, K3_CHUNK)
    dst3 = jnp.concatenate(
        [dst, dummy_dst]).reshape(NW, K3_NCHUNK, K3_CHUNK)

    hist = _k1(dst3)
    fpre, norm2d = _k2(hist.reshape(NW, NPAD // D, D), features)
    h0, h1 = _k3(src3, dst3, fpre)
    out = _k4(h0, h1, norm2d, features, initial_features, a_weight)
    return out
